# baseline (speedup 1.0000x reference)
import functools

import numpy as np
import jax
import jax.numpy as jnp
from jax import lax
from jax.experimental import pallas as pl
from jax.experimental.pallas import tpu as pltpu
from jax.experimental.pallas import tpu_sc as plsc

_BATCH, _BLOCKS, _HEADS, _SIZE = 16, 4096, 16, 7
_WIN = _SIZE * _SIZE
_POS = 2 * _SIZE - 1
_PP = _POS * _POS
_ROW = _HEADS * _WIN


def _coord_table(blocks, size):
    height = int(np.sqrt(float(blocks)))
    width = blocks // height
    pos_size = 2 * size - 1
    idx = np.arange(size)
    coords = (idx[:, None] * pos_size + idx).reshape(-1)
    bias_hh = np.concatenate(
        [idx[:size // 2], np.repeat(idx[size // 2], height - size + 1), idx[size // 2 + 1:]])
    bias_ww = np.concatenate(
        [idx[:size // 2], np.repeat(idx[size // 2], width - size + 1), idx[size // 2 + 1:]])
    bias_hw = bias_hh[:, None] * pos_size + bias_ww
    bc = (bias_hw[..., None] + coords).reshape(-1, size * size)[::-1]
    return np.ascontiguousarray(bc).astype(np.int32)


_BC = _coord_table(_BLOCKS, _SIZE)
_UC, _INV = np.unique(_BC, axis=0, return_inverse=True)
_U = int(_UC.shape[0])

_UIDX = (np.arange(_HEADS, dtype=np.int64)[None, :, None] * _PP
         + _UC[:, None, :].astype(np.int64)).reshape(_U * _ROW).astype(np.int32)

_OH = np.eye(_U, dtype=np.float32)[_INV]

_LANES = 16
_NW = 32
_NVEC = -(-(_U * _ROW) // _LANES)
_PER_W = -(-_NVEC // _NW)
_PAD_ELEMS = _NW * _PER_W * _LANES
_UIDX_PAD = np.zeros((_PAD_ELEMS,), np.int32)
_UIDX_PAD[:_U * _ROW] = _UIDX

@functools.lru_cache(maxsize=None)
def _make_ub_gather():
    mesh = plsc.VectorSubcoreMesh(core_axis_name="c", subcore_axis_name="s")

    @functools.partial(
        pl.kernel,
        out_type=jax.ShapeDtypeStruct((_PAD_ELEMS,), jnp.float32),
        mesh=mesh,
        compiler_params=pltpu.CompilerParams(needs_layout_passes=False),
        scratch_types=[
            pltpu.VMEM((_PER_W * _LANES,), jnp.int32),
            pltpu.VMEM((_PER_W * _LANES,), jnp.float32),
            pltpu.VMEM((_HEADS * _PP,), jnp.float32),
        ],
    )
    def _ub_gather(uidx_hbm, pb_hbm, out_hbm, idx_v, val_v, pb_v):
        wid = lax.axis_index("s") * 2 + lax.axis_index("c")
        base = wid * (_PER_W * _LANES)
        pltpu.sync_copy(pb_hbm, pb_v)
        pltpu.sync_copy(uidx_hbm.at[pl.ds(base, _PER_W * _LANES)], idx_v)
        for j in range(_PER_W):
            idx = idx_v[pl.ds(j * _LANES, _LANES)]
            val_v[pl.ds(j * _LANES, _LANES)] = plsc.load_gather(pb_v, [idx])
        pltpu.sync_copy(val_v, out_hbm.at[pl.ds(base, _PER_W * _LANES)])

    return _ub_gather


_CB = 512
_NCH = _BLOCKS // _CB
_PIECES = 8
_PB = _BATCH // _PIECES


def _add_body(in_ref, oh_ref, ub_ref, out_ref, bias_ref):
    @pl.when(pl.program_id(1) == 0)
    def _():
        bias_ref[...] = jnp.dot(oh_ref[...], ub_ref[...],
                                preferred_element_type=jnp.float32)

    out_ref[...] = in_ref[...] + bias_ref[...][None]


_add_call = pl.pallas_call(
    _add_body,
    grid=(_NCH, _PB),
    in_specs=[
        pl.BlockSpec((1, _CB, _ROW), lambda c, b: (b, c, 0)),
        pl.BlockSpec((_CB, _U), lambda c, b: (c, 0)),
        pl.BlockSpec((_U, _ROW), lambda c, b: (0, 0)),
    ],
    out_specs=pl.BlockSpec((1, _CB, _ROW), lambda c, b: (b, c, 0)),
    out_shape=jax.ShapeDtypeStruct((_PB, _BLOCKS, _ROW), jnp.float32),
    scratch_shapes=[pltpu.VMEM((_CB, _ROW), jnp.float32)],
    compiler_params=pltpu.CompilerParams(
        dimension_semantics=("arbitrary", "arbitrary")),
)


def kernel(inputs, pos_bias):
    ub_flat = _make_ub_gather()(jnp.asarray(_UIDX_PAD), pos_bias.reshape(-1))
    ub = ub_flat[:_U * _ROW].reshape(_U, _ROW)
    oh = jnp.asarray(_OH)
    outs = []
    for p in range(_PIECES):
        piece = inputs[p * _PB:(p + 1) * _PB].reshape(_PB, _BLOCKS, _ROW)
        o = _add_call(piece, oh, ub)
        outs.append(o.reshape(_PB, _BLOCKS, _HEADS, _WIN))
    return jnp.concatenate(outs, axis=0)


# speedup vs baseline: 11.3375x; 11.3375x over previous
import functools

import numpy as np
import jax
import jax.numpy as jnp
from jax import lax
from jax.experimental import pallas as pl
from jax.experimental.pallas import tpu as pltpu
from jax.experimental.pallas import tpu_sc as plsc

_BATCH, _BLOCKS, _HEADS, _SIZE = 16, 4096, 16, 7
_WIN = _SIZE * _SIZE
_POS = 2 * _SIZE - 1
_PP = _POS * _POS
_ROW = _HEADS * _WIN


def _coord_table(blocks, size):
    height = int(np.sqrt(float(blocks)))
    width = blocks // height
    pos_size = 2 * size - 1
    idx = np.arange(size)
    coords = (idx[:, None] * pos_size + idx).reshape(-1)
    bias_hh = np.concatenate(
        [idx[:size // 2], np.repeat(idx[size // 2], height - size + 1), idx[size // 2 + 1:]])
    bias_ww = np.concatenate(
        [idx[:size // 2], np.repeat(idx[size // 2], width - size + 1), idx[size // 2 + 1:]])
    bias_hw = bias_hh[:, None] * pos_size + bias_ww
    bc = (bias_hw[..., None] + coords).reshape(-1, size * size)[::-1]
    return np.ascontiguousarray(bc).astype(np.int32)


_BC = _coord_table(_BLOCKS, _SIZE)
_UC, _INV = np.unique(_BC, axis=0, return_inverse=True)
_U = int(_UC.shape[0])

_UIDX = (np.arange(_HEADS, dtype=np.int64)[None, :, None] * _PP
         + _UC.T[:, None, :].astype(np.int64)).reshape(_ROW * _U).astype(np.int32)

_OHT = np.eye(_U, dtype=np.float32)[_INV].T.copy()

_LANES = 16
_NW = 32
_NVEC = -(-(_U * _ROW) // _LANES)
_PER_W = -(-_NVEC // _NW)
_PAD_ELEMS = _NW * _PER_W * _LANES
_UIDX_PAD = np.zeros((_PAD_ELEMS,), np.int32)
_UIDX_PAD[:_U * _ROW] = _UIDX

@functools.lru_cache(maxsize=None)
def _make_ub_gather():
    mesh = plsc.VectorSubcoreMesh(core_axis_name="c", subcore_axis_name="s")

    @functools.partial(
        pl.kernel,
        out_type=jax.ShapeDtypeStruct((_PAD_ELEMS,), jnp.float32),
        mesh=mesh,
        compiler_params=pltpu.CompilerParams(needs_layout_passes=False),
        scratch_types=[
            pltpu.VMEM((_PER_W * _LANES,), jnp.int32),
            pltpu.VMEM((_PER_W * _LANES,), jnp.float32),
            pltpu.VMEM((_HEADS * _PP,), jnp.float32),
        ],
    )
    def _ub_gather(uidx_hbm, pb_hbm, out_hbm, idx_v, val_v, pb_v):
        wid = lax.axis_index("s") * 2 + lax.axis_index("c")
        base = wid * (_PER_W * _LANES)
        pltpu.sync_copy(pb_hbm, pb_v)
        pltpu.sync_copy(uidx_hbm.at[pl.ds(base, _PER_W * _LANES)], idx_v)
        for j in range(_PER_W):
            idx = idx_v[pl.ds(j * _LANES, _LANES)]
            val_v[pl.ds(j * _LANES, _LANES)] = plsc.load_gather(pb_v, [idx])
        pltpu.sync_copy(val_v, out_hbm.at[pl.ds(base, _PER_W * _LANES)])

    return _ub_gather


_CBK = 512
_NCH = _BLOCKS // _CBK


def _add_body(in_ref, ubt_ref, oht_ref, out_ref, bias_ref):
    @pl.when(pl.program_id(1) == 0)
    def _():
        bias_ref[...] = jnp.dot(ubt_ref[...], oht_ref[...],
                                preferred_element_type=jnp.float32)

    out_ref[...] = in_ref[...] + bias_ref[...][None]


_add_call = pl.pallas_call(
    _add_body,
    grid=(_NCH, _BATCH),
    in_specs=[
        pl.BlockSpec((1, _ROW, _CBK), lambda c, b: (b, 0, c)),
        pl.BlockSpec((_ROW, _U), lambda c, b: (0, 0)),
        pl.BlockSpec((_U, _CBK), lambda c, b: (0, c)),
    ],
    out_specs=pl.BlockSpec((1, _ROW, _CBK), lambda c, b: (b, 0, c)),
    out_shape=jax.ShapeDtypeStruct((_BATCH, _ROW, _BLOCKS), jnp.float32),
    scratch_shapes=[pltpu.VMEM((_ROW, _CBK), jnp.float32)],
    compiler_params=pltpu.CompilerParams(
        dimension_semantics=("arbitrary", "arbitrary")),
)


def kernel(inputs, pos_bias):
    ub_flat = _make_ub_gather()(jnp.asarray(_UIDX_PAD), pos_bias.reshape(-1))
    ubt = ub_flat[:_ROW * _U].reshape(_ROW, _U)
    xt = inputs.transpose(0, 3, 2, 1).reshape(_BATCH, _ROW, _BLOCKS)
    out = _add_call(xt, ubt, jnp.asarray(_OHT))
    return out.reshape(_BATCH, _WIN, _HEADS, _BLOCKS).transpose(0, 3, 2, 1)


# speedup vs baseline: 13.2004x; 1.1643x over previous
import functools

import numpy as np
import jax
import jax.numpy as jnp
from jax import lax
from jax.experimental import pallas as pl
from jax.experimental.pallas import tpu as pltpu
from jax.experimental.pallas import tpu_sc as plsc

_BATCH, _BLOCKS, _HEADS, _SIZE = 16, 4096, 16, 7
_WIN = _SIZE * _SIZE
_POS = 2 * _SIZE - 1
_PP = _POS * _POS
_ROW = _HEADS * _WIN


def _coord_table(blocks, size):
    height = int(np.sqrt(float(blocks)))
    width = blocks // height
    pos_size = 2 * size - 1
    idx = np.arange(size)
    coords = (idx[:, None] * pos_size + idx).reshape(-1)
    bias_hh = np.concatenate(
        [idx[:size // 2], np.repeat(idx[size // 2], height - size + 1), idx[size // 2 + 1:]])
    bias_ww = np.concatenate(
        [idx[:size // 2], np.repeat(idx[size // 2], width - size + 1), idx[size // 2 + 1:]])
    bias_hw = bias_hh[:, None] * pos_size + bias_ww
    bc = (bias_hw[..., None] + coords).reshape(-1, size * size)[::-1]
    return np.ascontiguousarray(bc).astype(np.int32)


_BC = _coord_table(_BLOCKS, _SIZE)
_UC, _INV = np.unique(_BC, axis=0, return_inverse=True)
_U = int(_UC.shape[0])

_UIDX = (np.arange(_HEADS, dtype=np.int64)[None, :, None] * _PP
         + _UC.T[:, None, :].astype(np.int64)).reshape(_ROW * _U).astype(np.int32)

_OHT = np.eye(_U, dtype=np.float32)[_INV].T.copy()

_LANES = 16
_NW = 32
_NVEC = -(-(_U * _ROW) // _LANES)
_PER_W = -(-_NVEC // _NW)
_PAD_ELEMS = _NW * _PER_W * _LANES
_UIDX_PAD = np.zeros((_PAD_ELEMS,), np.int32)
_UIDX_PAD[:_U * _ROW] = _UIDX

@functools.lru_cache(maxsize=None)
def _make_ub_gather():
    mesh = plsc.VectorSubcoreMesh(core_axis_name="c", subcore_axis_name="s")

    @functools.partial(
        pl.kernel,
        out_type=jax.ShapeDtypeStruct((_PAD_ELEMS,), jnp.float32),
        mesh=mesh,
        compiler_params=pltpu.CompilerParams(needs_layout_passes=False),
        scratch_types=[
            pltpu.VMEM((_PER_W * _LANES,), jnp.int32),
            pltpu.VMEM((_PER_W * _LANES,), jnp.float32),
            pltpu.VMEM((_HEADS * _PP,), jnp.float32),
        ],
    )
    def _ub_gather(uidx_hbm, pb_hbm, out_hbm, idx_v, val_v, pb_v):
        wid = lax.axis_index("s") * 2 + lax.axis_index("c")
        base = wid * (_PER_W * _LANES)
        pltpu.sync_copy(pb_hbm, pb_v)
        pltpu.sync_copy(uidx_hbm.at[pl.ds(base, _PER_W * _LANES)], idx_v)
        for j in range(_PER_W):
            idx = idx_v[pl.ds(j * _LANES, _LANES)]
            val_v[pl.ds(j * _LANES, _LANES)] = plsc.load_gather(pb_v, [idx])
        pltpu.sync_copy(val_v, out_hbm.at[pl.ds(base, _PER_W * _LANES)])

    return _ub_gather


_CBK = 1024
_NCH = _BLOCKS // _CBK


def _add_body(in_ref, ubt_ref, oht_ref, out_ref, bias_ref):
    @pl.when(pl.program_id(1) == 0)
    def _():
        bias_ref[...] = jnp.dot(ubt_ref[...], oht_ref[...],
                                preferred_element_type=jnp.float32)

    out_ref[...] = in_ref[...] + bias_ref[...][None]


_add_call = pl.pallas_call(
    _add_body,
    grid=(_NCH, _BATCH),
    in_specs=[
        pl.BlockSpec((1, _ROW, _CBK), lambda c, b: (b, 0, c)),
        pl.BlockSpec((_ROW, _U), lambda c, b: (0, 0)),
        pl.BlockSpec((_U, _CBK), lambda c, b: (0, c)),
    ],
    out_specs=pl.BlockSpec((1, _ROW, _CBK), lambda c, b: (b, 0, c)),
    out_shape=jax.ShapeDtypeStruct((_BATCH, _ROW, _BLOCKS), jnp.float32),
    scratch_shapes=[pltpu.VMEM((_ROW, _CBK), jnp.float32)],
    compiler_params=pltpu.CompilerParams(
        dimension_semantics=("arbitrary", "arbitrary")),
)


def kernel(inputs, pos_bias):
    ub_flat = _make_ub_gather()(jnp.asarray(_UIDX_PAD), pos_bias.reshape(-1))
    ubt = ub_flat[:_ROW * _U].reshape(_ROW, _U)
    xt = inputs.transpose(0, 3, 2, 1).reshape(_BATCH, _ROW, _BLOCKS)
    out = _add_call(xt, ubt, jnp.asarray(_OHT))
    return out.reshape(_BATCH, _WIN, _HEADS, _BLOCKS).transpose(0, 3, 2, 1)


# speedup vs baseline: 13.5490x; 1.0264x over previous
import functools

import numpy as np
import jax
import jax.numpy as jnp
from jax import lax
from jax.experimental import pallas as pl
from jax.experimental.pallas import tpu as pltpu
from jax.experimental.pallas import tpu_sc as plsc

_BATCH, _BLOCKS, _HEADS, _SIZE = 16, 4096, 16, 7
_WIN = _SIZE * _SIZE
_POS = 2 * _SIZE - 1
_PP = _POS * _POS
_ROW = _HEADS * _WIN


def _coord_table(blocks, size):
    height = int(np.sqrt(float(blocks)))
    width = blocks // height
    pos_size = 2 * size - 1
    idx = np.arange(size)
    coords = (idx[:, None] * pos_size + idx).reshape(-1)
    bias_hh = np.concatenate(
        [idx[:size // 2], np.repeat(idx[size // 2], height - size + 1), idx[size // 2 + 1:]])
    bias_ww = np.concatenate(
        [idx[:size // 2], np.repeat(idx[size // 2], width - size + 1), idx[size // 2 + 1:]])
    bias_hw = bias_hh[:, None] * pos_size + bias_ww
    bc = (bias_hw[..., None] + coords).reshape(-1, size * size)[::-1]
    return np.ascontiguousarray(bc).astype(np.int32)


_BC = _coord_table(_BLOCKS, _SIZE)
_UC, _INV = np.unique(_BC, axis=0, return_inverse=True)
_U = int(_UC.shape[0])

_UIDX = (np.arange(_HEADS, dtype=np.int64)[None, :, None] * _PP
         + _UC.T[:, None, :].astype(np.int64)).reshape(_ROW * _U).astype(np.int32)

_OHT = np.eye(_U, dtype=np.float32)[_INV].T.copy()

_LANES = 16
_NW = 32
_NVEC = -(-(_U * _ROW) // _LANES)
_PER_W = -(-_NVEC // _NW)
_PAD_ELEMS = _NW * _PER_W * _LANES
_UIDX_PAD = np.zeros((_PAD_ELEMS,), np.int32)
_UIDX_PAD[:_U * _ROW] = _UIDX

@functools.lru_cache(maxsize=None)
def _make_ub_gather():
    mesh = plsc.VectorSubcoreMesh(core_axis_name="c", subcore_axis_name="s")

    @functools.partial(
        pl.kernel,
        out_type=jax.ShapeDtypeStruct((_PAD_ELEMS,), jnp.float32),
        mesh=mesh,
        compiler_params=pltpu.CompilerParams(needs_layout_passes=False),
        scratch_types=[
            pltpu.VMEM((_PER_W * _LANES,), jnp.int32),
            pltpu.VMEM((_PER_W * _LANES,), jnp.float32),
            pltpu.VMEM((_HEADS * _PP,), jnp.float32),
        ],
    )
    def _ub_gather(uidx_hbm, pb_hbm, out_hbm, idx_v, val_v, pb_v):
        wid = lax.axis_index("s") * 2 + lax.axis_index("c")
        base = wid * (_PER_W * _LANES)
        pltpu.sync_copy(pb_hbm, pb_v)
        pltpu.sync_copy(uidx_hbm.at[pl.ds(base, _PER_W * _LANES)], idx_v)
        for j in range(_PER_W):
            idx = idx_v[pl.ds(j * _LANES, _LANES)]
            val_v[pl.ds(j * _LANES, _LANES)] = plsc.load_gather(pb_v, [idx])
        pltpu.sync_copy(val_v, out_hbm.at[pl.ds(base, _PER_W * _LANES)])

    return _ub_gather


_CBK = 2048
_NCH = _BLOCKS // _CBK


def _add_body(in_ref, ubt_ref, oht_ref, out_ref, bias_ref):
    @pl.when(pl.program_id(1) == 0)
    def _():
        bias_ref[...] = jnp.dot(ubt_ref[...], oht_ref[...],
                                preferred_element_type=jnp.float32)

    out_ref[...] = in_ref[...] + bias_ref[...][None]


_add_call = pl.pallas_call(
    _add_body,
    grid=(_NCH, _BATCH),
    in_specs=[
        pl.BlockSpec((1, _ROW, _CBK), lambda c, b: (b, 0, c)),
        pl.BlockSpec((_ROW, _U), lambda c, b: (0, 0)),
        pl.BlockSpec((_U, _CBK), lambda c, b: (0, c)),
    ],
    out_specs=pl.BlockSpec((1, _ROW, _CBK), lambda c, b: (b, 0, c)),
    out_shape=jax.ShapeDtypeStruct((_BATCH, _ROW, _BLOCKS), jnp.float32),
    scratch_shapes=[pltpu.VMEM((_ROW, _CBK), jnp.float32)],
    compiler_params=pltpu.CompilerParams(
        dimension_semantics=("arbitrary", "arbitrary")),
)


def kernel(inputs, pos_bias):
    ub_flat = _make_ub_gather()(jnp.asarray(_UIDX_PAD), pos_bias.reshape(-1))
    ubt = ub_flat[:_ROW * _U].reshape(_ROW, _U)
    xt = inputs.transpose(0, 3, 2, 1).reshape(_BATCH, _ROW, _BLOCKS)
    out = _add_call(xt, ubt, jnp.asarray(_OHT))
    return out.reshape(_BATCH, _WIN, _HEADS, _BLOCKS).transpose(0, 3, 2, 1)
